# Initial kernel scaffold; baseline (speedup 1.0000x reference)
import jax
import jax.numpy as jnp
from jax.experimental import pallas as pl


def kernel(X, W, b, M):
    raise NotImplementedError("write your pallas kernel here")


# speedup vs baseline: 1.1472x; 1.1472x over previous
import functools

import jax
import jax.numpy as jnp
from jax.experimental import pallas as pl
from jax.experimental.pallas import tpu as pltpu

B = 16384
IN_SIZE = 128
HIDD = 64
MEM = 4096

BB = 512


def _body(x_ref, wt_ref, b_ref, m_ref, o_ref):
    x = x_ref[...]
    wt = wt_ref[...]
    m = m_ref[...]

    h = jnp.dot(x, wt, preferred_element_type=jnp.float32) + b_ref[...]
    h = jnp.where(h >= 0, h, 0.01 * h)

    inner = jax.lax.dot_general(h, m, (((1,), (1,)), ((), ())),
                                preferred_element_type=jnp.float32)
    k_n = jnp.sqrt(jnp.sum(h * h, axis=-1, keepdims=True))
    m_n = jnp.sqrt(jnp.sum(m * m, axis=-1))[None, :]
    coss = inner / (k_n * m_n + 1e-8)

    e = jnp.exp(coss - jnp.max(coss, axis=-1, keepdims=True))
    w_r = e / jnp.sum(e, axis=-1, keepdims=True)

    read = jnp.dot(w_r, m, preferred_element_type=jnp.float32)

    o_ref[:, :HIDD] = h
    o_ref[:, HIDD:] = read


@functools.partial(jax.jit, static_argnames=())
def kernel(X, W, b, M):
    wt = W.T
    b2 = b.reshape(1, HIDD)
    grid = (B // BB,)
    out = pl.pallas_call(
        _body,
        grid=grid,
        in_specs=[
            pl.BlockSpec((BB, IN_SIZE), lambda i: (i, 0)),
            pl.BlockSpec((IN_SIZE, HIDD), lambda i: (0, 0)),
            pl.BlockSpec((1, HIDD), lambda i: (0, 0)),
            pl.BlockSpec((MEM, HIDD), lambda i: (0, 0)),
        ],
        out_specs=pl.BlockSpec((BB, 2 * HIDD), lambda i: (i, 0)),
        out_shape=jax.ShapeDtypeStruct((B, 2 * HIDD), jnp.float32),
        compiler_params=pltpu.CompilerParams(
            dimension_semantics=("arbitrary",),
        ),
    )(X, wt, b2, M)
    return out


# speedup vs baseline: 2.5168x; 2.1939x over previous
import functools

import jax
import jax.numpy as jnp
from jax.experimental import pallas as pl
from jax.experimental.pallas import tpu as pltpu

B = 16384
IN_SIZE = 128
HIDD = 64
MEM = 4096

BB = 512


def _body(x_ref, wt_ref, b_ref, m_ref, o_ref, mn_ref, mext_ref):
    @pl.when(pl.program_id(0) == 0)
    def _init():
        m = m_ref[...]
        ss = jnp.sum(m * m, axis=-1, keepdims=True)
        mn_ref[...] = m * jax.lax.rsqrt(jnp.maximum(ss, 1e-30))
        mext_ref[:, :HIDD] = m
        lane = jax.lax.broadcasted_iota(jnp.int32, (MEM, 128 - HIDD), 1)
        mext_ref[:, HIDD:] = jnp.where(lane == 0, 1.0, 0.0)

    x = x_ref[...]

    h = jnp.dot(x, wt_ref[...], preferred_element_type=jnp.float32) + b_ref[...]
    h = jnp.where(h >= 0, h, 0.01 * h)

    hs = jnp.sum(h * h, axis=-1, keepdims=True)
    hn = h * jax.lax.rsqrt(jnp.maximum(hs, 1e-30))

    coss = jax.lax.dot_general(hn, mn_ref[...], (((1,), (1,)), ((), ())),
                               preferred_element_type=jnp.float32)
    e = jnp.exp(coss)

    rext = jnp.dot(e, mext_ref[...], preferred_element_type=jnp.float32)
    read = rext[:, :HIDD] / rext[:, HIDD:HIDD + 1]

    o_ref[:, :HIDD] = h
    o_ref[:, HIDD:] = read


@functools.partial(jax.jit, static_argnames=())
def kernel(X, W, b, M):
    wt = W.T
    b2 = b.reshape(1, HIDD)
    grid = (B // BB,)
    out = pl.pallas_call(
        _body,
        grid=grid,
        in_specs=[
            pl.BlockSpec((BB, IN_SIZE), lambda i: (i, 0)),
            pl.BlockSpec((IN_SIZE, HIDD), lambda i: (0, 0)),
            pl.BlockSpec((1, HIDD), lambda i: (0, 0)),
            pl.BlockSpec((MEM, HIDD), lambda i: (0, 0)),
        ],
        out_specs=pl.BlockSpec((BB, 2 * HIDD), lambda i: (i, 0)),
        out_shape=jax.ShapeDtypeStruct((B, 2 * HIDD), jnp.float32),
        scratch_shapes=[
            pltpu.VMEM((MEM, HIDD), jnp.float32),
            pltpu.VMEM((MEM, 128), jnp.float32),
        ],
        compiler_params=pltpu.CompilerParams(
            dimension_semantics=("arbitrary",),
        ),
    )(X, wt, b2, M)
    return out
